# baseline (speedup 1.0000x reference)
import jax
import jax.numpy as jnp
from jax import lax
from jax.experimental import pallas as pl
from jax.experimental.pallas import tpu as pltpu
from jax.experimental.pallas import tpu_sc as plsc

_BATCH = 16384
_N_FIELDS = 26
_NW = 32
_BPW = _BATCH // _NW
_IPW = _BPW * _N_FIELDS
_CHUNK = 128
_CPF = _BPW // _CHUNK
_L = 16


def _sc_body(idxT_hbm, wT_hbm, out_hbm, idx_v, g_v, out_v, sem):
    wid = lax.axis_index("s") * 2 + lax.axis_index("c")
    col0 = pl.multiple_of(wid * _BPW, _BPW)

    pltpu.sync_copy(idxT_hbm.at[:, pl.ds(col0, _BPW)], idx_v)

    w_flat = wT_hbm.at[0]

    def gather_step(f, _):
        for c in range(_CPF):
            off = pl.multiple_of(f * _BPW + c * _CHUNK, _CHUNK)
            pltpu.async_copy(
                w_flat.at[idx_v.at[f, pl.ds(c * _CHUNK, _CHUNK)]],
                g_v.at[pl.ds(off, _CHUNK)],
                sem,
            )
        return 0

    lax.fori_loop(0, _N_FIELDS, gather_step, 0)

    pltpu.make_async_copy(
        w_flat.at[pl.ds(0, _IPW)], g_v, sem
    ).wait()

    def reduce_step(g, _):
        base = pl.multiple_of(g * _L, _L)
        acc = g_v[pl.ds(base, _L)]
        for f in range(1, _N_FIELDS):
            acc = acc + g_v[pl.ds(f * _BPW + base, _L)]
        out_v[pl.ds(base, _L)] = acc
        return 0

    lax.fori_loop(0, _BPW // _L, reduce_step, 0)

    pltpu.sync_copy(out_v, out_hbm.at[0, pl.ds(col0, _BPW)])


@jax.jit
def _run(idxT, wT):
    mesh = plsc.VectorSubcoreMesh(core_axis_name="c", subcore_axis_name="s")
    return pl.kernel(
        _sc_body,
        out_type=jax.ShapeDtypeStruct((1, _BATCH), jnp.float32),
        mesh=mesh,
        compiler_params=pltpu.CompilerParams(needs_layout_passes=False),
        scratch_types=[
            pltpu.VMEM((_N_FIELDS, _BPW), jnp.int32),
            pltpu.VMEM((_IPW,), jnp.float32),
            pltpu.VMEM((_BPW,), jnp.float32),
            pltpu.SemaphoreType.DMA,
        ],
    )(idxT, wT)


def kernel(inputs, w):
    out = _run(inputs.T, w.T)
    return out.reshape(_BATCH, 1)


# speedup vs baseline: 1.0050x; 1.0050x over previous
import jax
import jax.numpy as jnp
from jax import lax
from jax.experimental import pallas as pl
from jax.experimental.pallas import tpu as pltpu
from jax.experimental.pallas import tpu_sc as plsc

_BATCH = 16384
_N_FIELDS = 26
_NW = 32
_BPW = _BATCH // _NW
_IPW = _BPW * _N_FIELDS
_CHUNK = 128
_CPF = _BPW // _CHUNK
_L = 16


def _sc_body(idxT_hbm, wT_hbm, out_hbm, idx_v, g_v, out_v, sem):
    wid = lax.axis_index("s") * 2 + lax.axis_index("c")
    col0 = pl.multiple_of(wid * _BPW, _BPW)

    pltpu.sync_copy(idxT_hbm.at[:, pl.ds(col0, _BPW)], idx_v)

    w_flat = wT_hbm.at[0]

    def gather_step(f, _):
        for c in range(_CPF):
            off = pl.multiple_of(f * _BPW + c * _CHUNK, _CHUNK)
            pltpu.async_copy(
                w_flat.at[idx_v.at[f, pl.ds(c * _CHUNK, _CHUNK)]],
                g_v.at[pl.ds(off, _CHUNK)],
                sem,
            )
        return 0

    lax.fori_loop(0, _N_FIELDS, gather_step, 0)

    pltpu.make_async_copy(
        w_flat.at[pl.ds(0, _IPW)], g_v, sem
    ).wait()

    def reduce_step(g, _):
        base = pl.multiple_of(g * _L, _L)
        acc = g_v[pl.ds(base, _L)]
        for f in range(1, _N_FIELDS):
            acc = acc + g_v[pl.ds(f * _BPW + base, _L)]
        out_v[pl.ds(base, _L)] = acc
        return 0

    lax.fori_loop(0, _BPW // _L, reduce_step, 0)

    pltpu.sync_copy(out_v, out_hbm.at[0, pl.ds(col0, _BPW)])


@jax.jit
def _run(idxT, wT):
    mesh = plsc.VectorSubcoreMesh(core_axis_name="c", subcore_axis_name="s")
    return pl.kernel(
        _sc_body,
        out_type=jax.ShapeDtypeStruct((1, _BATCH), jnp.float32),
        mesh=mesh,
        scratch_types=[
            pltpu.VMEM((_N_FIELDS, _BPW), jnp.int32),
            pltpu.VMEM((_IPW,), jnp.float32),
            pltpu.VMEM((_BPW,), jnp.float32),
            pltpu.SemaphoreType.DMA,
        ],
    )(idxT, wT)


def kernel(inputs, w):
    out = _run(inputs.T, w.T)
    return out.reshape(_BATCH, 1)
